# Initial kernel scaffold; baseline (speedup 1.0000x reference)
import jax
import jax.numpy as jnp
from jax.experimental import pallas as pl


def kernel(ids, embs):
    raise NotImplementedError("write your pallas kernel here")


# speedup vs baseline: 1.0666x; 1.0666x over previous
import functools

import jax
import jax.numpy as jnp
from jax import lax
from jax.experimental import pallas as pl
from jax.experimental.pallas import tpu as pltpu
from jax.experimental.pallas import tpu_sc as plsc

_LANES = 16
_CSUB = 128


def _build_gather(num_flat, dim, chunk, interpret=False):
    nc, ns = 2, 16
    nw = nc * ns
    assert num_flat % (nw * chunk) == 0
    per_w = num_flat // nw
    n_chunks = per_w // chunk
    n_sub = chunk // _CSUB
    n_grp = chunk // _LANES
    id_rows = chunk // _CSUB

    mesh = plsc.VectorSubcoreMesh(
        core_axis_name="c", subcore_axis_name="s", num_cores=nc, num_subcores=ns
    )

    @functools.partial(
        pl.kernel,
        out_type=jax.ShapeDtypeStruct((num_flat, dim), jnp.float32),
        mesh=mesh,
        interpret=interpret,
        compiler_params=pltpu.CompilerParams(
            use_tc_tiling_on_sc=False, needs_layout_passes=False
        ),
        scratch_types=[
            pltpu.VMEM((id_rows, _CSUB), jnp.int32),
            pltpu.VMEM((id_rows, _CSUB), jnp.int32),
            pltpu.VMEM((chunk, dim), jnp.float32),
            pltpu.SemaphoreType.DMA,
        ],
    )
    def body(ids_hbm, table_hbm, out_hbm, idx_raw, idx_adj, rows, sem):
        wid = lax.axis_index("s") * nc + lax.axis_index("c")
        zeros = jnp.zeros((_LANES,), jnp.float32)
        lane_iota = lax.iota(jnp.int32, _LANES)
        ones_i = jnp.full((_LANES,), 1, jnp.int32)
        zeros_i = jnp.zeros((_LANES,), jnp.int32)

        def chunk_body(g, carry):
            id_row0 = wid * (per_w // _CSUB) + g * id_rows
            out_row0 = wid * per_w + g * chunk

            pltpu.sync_copy(ids_hbm.at[pl.ds(id_row0, id_rows)], idx_raw)

            for r in range(id_rows):
                for c in range(_CSUB // _LANES):
                    v = idx_raw[r, pl.ds(c * _LANES, _LANES)]
                    idx_adj[r, pl.ds(c * _LANES, _LANES)] = jnp.maximum(
                        v - ones_i, zeros_i
                    )

            descs = [
                pltpu.async_copy(
                    table_hbm.at[idx_adj.at[r]],
                    rows.at[pl.ds(r * _CSUB, _CSUB)],
                    sem,
                )
                for r in range(n_sub)
            ]
            for d in descs:
                d.wait()

            for grp in range(n_grp):
                r, c = grp // (_CSUB // _LANES), grp % (_CSUB // _LANES)
                v = idx_raw[r, pl.ds(c * _LANES, _LANES)]
                mask = v == zeros_i
                cnt = jnp.sum(jnp.where(mask, ones_i, zeros_i))

                @pl.when(cnt > 0)
                def _fix(grp=grp, mask=mask):
                    row_idx = jnp.full((_LANES,), grp * _LANES, jnp.int32) + lane_iota
                    for k in range(dim):
                        plsc.store_scatter(
                            rows,
                            [row_idx, jnp.full((_LANES,), k, jnp.int32)],
                            zeros,
                            mask=mask,
                        )

            pltpu.sync_copy(rows, out_hbm.at[pl.ds(out_row0, chunk)])
            return carry

        lax.fori_loop(0, n_chunks, chunk_body, 0)

    return body


def kernel(ids, embs):
    batch, hist = ids.shape
    vocab, dim = embs.shape
    num_flat = batch * hist
    ids_flat = ids.reshape(num_flat).astype(jnp.int32).reshape(-1, _CSUB)
    gather = _build_gather(num_flat, dim, chunk=1024)
    out_flat = gather(ids_flat, embs)
    return out_flat.reshape(batch, hist, dim)


# speedup vs baseline: 1.5911x; 1.4918x over previous
import functools

import jax
import jax.numpy as jnp
from jax import lax
from jax.experimental import pallas as pl
from jax.experimental.pallas import tpu as pltpu
from jax.experimental.pallas import tpu_sc as plsc

_LANES = 16


def _group_offsets(hist):
    offs = list(range(0, hist - _LANES + 1, _LANES))
    if offs[-1] + _LANES < hist:
        offs.append(hist - _LANES)
    return offs


def _build_lookup(batch, hist, dim, rows_per_chunk):
    nc, ns = 2, 16
    nw = nc * ns
    assert batch % (nw * rows_per_chunk) == 0
    rows_per_w = batch // nw
    n_chunks = rows_per_w // rows_per_chunk
    offs = _group_offsets(hist)

    mesh = plsc.VectorSubcoreMesh(
        core_axis_name="c", subcore_axis_name="s", num_cores=nc, num_subcores=ns
    )

    @functools.partial(
        pl.kernel,
        out_type=jax.ShapeDtypeStruct((batch, hist, dim), jnp.float32),
        mesh=mesh,
        compiler_params=pltpu.CompilerParams(
            use_tc_tiling_on_sc=False, needs_layout_passes=False
        ),
        scratch_types=[
            pltpu.VMEM((rows_per_chunk, hist), jnp.int32),
            pltpu.VMEM((rows_per_chunk, hist), jnp.int32),
            pltpu.VMEM((rows_per_chunk, hist, dim), jnp.float32),
            pltpu.SemaphoreType.DMA,
        ],
    )
    def body(ids_hbm, table_hbm, out_hbm, idx_raw, idx_adj, rows, sem):
        wid = lax.axis_index("s") * nc + lax.axis_index("c")
        zeros_f = jnp.zeros((_LANES,), jnp.float32)
        lane_iota = lax.iota(jnp.int32, _LANES)
        ones_i = jnp.full((_LANES,), 1, jnp.int32)
        zeros_i = jnp.zeros((_LANES,), jnp.int32)

        def chunk_body(g, carry):
            brow0 = wid * rows_per_w + g * rows_per_chunk

            pltpu.sync_copy(ids_hbm.at[pl.ds(brow0, rows_per_chunk)], idx_raw)

            for r in range(rows_per_chunk):
                for c in offs:
                    v = idx_raw[r, pl.ds(c, _LANES)]
                    idx_adj[r, pl.ds(c, _LANES)] = jnp.maximum(v - ones_i, zeros_i)

            descs = [
                pltpu.async_copy(
                    table_hbm.at[idx_adj.at[r]],
                    rows.at[r],
                    sem,
                )
                for r in range(rows_per_chunk)
            ]
            for d in descs:
                d.wait()

            for r in range(rows_per_chunk):
                for c in offs:
                    v = idx_raw[r, pl.ds(c, _LANES)]
                    mask = v == zeros_i
                    cnt = jnp.sum(jnp.where(mask, ones_i, zeros_i))

                    @pl.when(cnt > 0)
                    def _fix(r=r, c=c, mask=mask):
                        row_i = jnp.full((_LANES,), r, jnp.int32)
                        col_i = jnp.full((_LANES,), c, jnp.int32) + lane_iota
                        for k in range(dim):
                            plsc.store_scatter(
                                rows,
                                [row_i, col_i, jnp.full((_LANES,), k, jnp.int32)],
                                zeros_f,
                                mask=mask,
                            )

            pltpu.sync_copy(rows, out_hbm.at[pl.ds(brow0, rows_per_chunk)])
            return carry

        lax.fori_loop(0, n_chunks, chunk_body, 0)

    return body


def kernel(ids, embs):
    batch, hist = ids.shape
    vocab, dim = embs.shape
    lookup = _build_lookup(batch, hist, dim, rows_per_chunk=16)
    return lookup(ids.astype(jnp.int32), embs)
